# baseline (speedup 1.0000x reference)
import jax
import jax.numpy as jnp
from jax.experimental import pallas as pl
from jax.experimental.pallas import tpu as pltpu

_B = 128
_NE = 64
_NUP = 32
_NN = 16
_D = 256
_S = 32
_L = 3


def _gnn_body(r_ref, rT_ref, RT_ref, emb_ref, hn_ref,
              Wf_ne_ref, bf_ne_ref, Wf_same_ref, bf_same_ref,
              Wf_anti_ref, bf_anti_ref,
              Ww_ne_ref, Ww_same_ref, Ww_anti_ref,
              Wh_ne_ref, Wh_ee_ref, Wu_ref, bu_ref,
              out_ref):
    f32 = jnp.float32

    r2d = r_ref[0]
    rT2d = rT_ref[0]
    RT2d = RT_ref[0]

    diff_ee = []
    diff_ne = []
    s_ee = jnp.full((_NE, _NE), 1e-12, f32)
    s_ne = jnp.full((_NE, _NN), 1e-12, f32)
    for k in range(3):
        col = r2d[:, k:k + 1]
        dee = col - rT2d[k:k + 1, :]
        dne = col - RT2d[k:k + 1, :]
        diff_ee.append(dee)
        diff_ne.append(dne)
        s_ee = s_ee + dee * dee
        s_ne = s_ne + dne * dne
    d_ee = jnp.sqrt(s_ee)
    d_ne = jnp.sqrt(s_ne)

    feats_ee = [d_ee] + diff_ee
    feats_ne = [d_ne] + diff_ne

    def edge_embed(feats, Wf_ref, bf_ref):
        acc = jnp.broadcast_to(bf_ref[0:1, :][None, :, :],
                               (feats[0].shape[0], feats[0].shape[1], _S))
        for k in range(4):
            acc = acc + feats[k][:, :, None] * Wf_ref[k:k + 1, :][None, :, :]
        return jnp.tanh(acc)

    e_same3 = edge_embed(feats_ee, Wf_same_ref, bf_same_ref)
    e_anti3 = edge_embed(feats_ee, Wf_anti_ref, bf_anti_ref)
    e_ne3 = edge_embed(feats_ne, Wf_ne_ref, bf_ne_ref)

    ii = jax.lax.broadcasted_iota(jnp.int32, (_NE, _NE), 0)
    jj = jax.lax.broadcasted_iota(jnp.int32, (_NE, _NE), 1)
    spin_eq = (ii < _NUP) == (jj < _NUP)
    same_m = jnp.where(spin_eq & (ii != jj), 1.0, 0.0).astype(f32)
    anti_m = jnp.where(spin_eq, 0.0, 1.0).astype(f32)

    e_same = e_same3.reshape(_NE * _NE, _S)
    e_anti = e_anti3.reshape(_NE * _NE, _S)
    e_ne = e_ne3.reshape(_NE * _NN, _S)

    w_same = jnp.tanh(jnp.dot(e_same, Ww_same_ref[...],
                              preferred_element_type=f32))
    w_anti = jnp.tanh(jnp.dot(e_anti, Ww_anti_ref[...],
                              preferred_element_type=f32))
    w_ne = jnp.tanh(jnp.dot(e_ne, Ww_ne_ref[...],
                            preferred_element_type=f32))

    w_same3 = w_same.reshape(_NE, _NE, _L * _S) * same_m[:, :, None]
    w_anti3 = w_anti.reshape(_NE, _NE, _L * _S) * anti_m[:, :, None]
    w_ne3 = w_ne.reshape(_NE, _NN, _L * _S)

    row = jax.lax.broadcasted_iota(jnp.int32, (_NE, 1), 0)
    h = jnp.where(row < _NUP, emb_ref[0:1, :], emb_ref[1:2, :])
    hn = hn_ref[...]

    for l in range(_L):
        lo = l * _S
        hs_ee = jnp.tanh(jnp.dot(h, Wh_ee_ref[l],
                                 preferred_element_type=f32))
        hs_same = hs_ee[:, :_S]
        hs_anti = hs_ee[:, _S:]
        hs_ne = jnp.tanh(jnp.dot(hn, Wh_ne_ref[l],
                                 preferred_element_type=f32))

        z_ne = jnp.sum(w_ne3[:, :, lo:lo + _S] * hs_ne[None, :, :], axis=1)
        z_same = jnp.sum(w_same3[:, :, lo:lo + _S] * hs_same[None, :, :],
                         axis=1)
        z_anti = jnp.sum(w_anti3[:, :, lo:lo + _S] * hs_anti[None, :, :],
                         axis=1)
        z = jnp.concatenate([z_ne, z_same, z_anti], axis=-1)
        h = h + jnp.tanh(jnp.dot(z, Wu_ref[l],
                                 preferred_element_type=f32) + bu_ref[l])

    out_ref[0] = h


def kernel(r, R, elec_emb, nuc_emb,
           Wfeat_ne, bfeat_ne, Wfeat_same, bfeat_same, Wfeat_anti, bfeat_anti,
           Ww_ne_0, Wh_ne_0, Ww_same_0, Wh_same_0, Ww_anti_0, Wh_anti_0,
           Wu_0, bu_0,
           Ww_ne_1, Wh_ne_1, Ww_same_1, Wh_same_1, Ww_anti_1, Wh_anti_1,
           Wu_1, bu_1,
           Ww_ne_2, Wh_ne_2, Ww_same_2, Wh_same_2, Ww_anti_2, Wh_anti_2,
           Wu_2, bu_2):
    f32 = jnp.float32
    rT = jnp.swapaxes(r, 1, 2)
    RT = jnp.swapaxes(R, 1, 2)

    Ww_same = jnp.concatenate([Ww_same_0, Ww_same_1, Ww_same_2], axis=1)
    Ww_anti = jnp.concatenate([Ww_anti_0, Ww_anti_1, Ww_anti_2], axis=1)
    Ww_ne = jnp.concatenate([Ww_ne_0, Ww_ne_1, Ww_ne_2], axis=1)
    Wh_ee = jnp.stack([
        jnp.concatenate([Wh_same_0, Wh_anti_0], axis=1),
        jnp.concatenate([Wh_same_1, Wh_anti_1], axis=1),
        jnp.concatenate([Wh_same_2, Wh_anti_2], axis=1),
    ])
    Wh_ne = jnp.stack([Wh_ne_0, Wh_ne_1, Wh_ne_2])
    Wu = jnp.stack([Wu_0, Wu_1, Wu_2])
    bu = jnp.stack([bu_0, bu_1, bu_2]).reshape(_L, 1, _D)
    bf_ne = bfeat_ne.reshape(1, _S)
    bf_same = bfeat_same.reshape(1, _S)
    bf_anti = bfeat_anti.reshape(1, _S)

    def const_spec(x):
        nd = x.ndim
        return pl.BlockSpec(x.shape, lambda b, _n=nd: (0,) * _n)

    in_specs = [
        pl.BlockSpec((1, _NE, 3), lambda b: (b, 0, 0)),
        pl.BlockSpec((1, 3, _NE), lambda b: (b, 0, 0)),
        pl.BlockSpec((1, 3, _NN), lambda b: (b, 0, 0)),
    ] + [const_spec(x) for x in (
        elec_emb, nuc_emb,
        Wfeat_ne, bf_ne, Wfeat_same, bf_same, Wfeat_anti, bf_anti,
        Ww_ne, Ww_same, Ww_anti, Wh_ne, Wh_ee, Wu, bu)]

    out = pl.pallas_call(
        _gnn_body,
        grid=(_B,),
        in_specs=in_specs,
        out_specs=pl.BlockSpec((1, _NE, _D), lambda b: (b, 0, 0)),
        out_shape=jax.ShapeDtypeStruct((_B, _NE, _D), f32),
        compiler_params=pltpu.CompilerParams(
            dimension_semantics=("parallel",)),
    )(r, rT, RT, elec_emb, nuc_emb,
      Wfeat_ne, bf_ne, Wfeat_same, bf_same, Wfeat_anti, bf_anti,
      Ww_ne, Ww_same, Ww_anti, Wh_ne, Wh_ee, Wu, bu)
    return out


# speedup vs baseline: 1.8045x; 1.8045x over previous
import jax
import jax.numpy as jnp
from jax.experimental import pallas as pl
from jax.experimental.pallas import tpu as pltpu

_B = 128
_NE = 64
_NUP = 32
_NN = 16
_D = 256
_S = 32
_L = 3


def _gnn_body(r_ref, rT_ref, RT_ref,
              mask_ref, h0_ref, hn_ref,
              Wf0_ref, Wdir_all_ref, Wdir_ee_ref, Wfne14_ref, bf_ref,
              BD_ref, Wwne_ref, Whne_ref, Whee_ref, Wu_ref, bu_ref,
              out_ref):
    f32 = jnp.float32

    r2d = r_ref[0]
    rT2d = rT_ref[0]
    RT2d = RT_ref[0]

    s_ee = jnp.full((_NE, _NE), 1e-12, f32)
    s_ne = jnp.full((_NE, _NN), 1e-12, f32)
    for k in range(3):
        col = r2d[:, k:k + 1]
        dee = col - rT2d[k:k + 1, :]
        dne = col - RT2d[k:k + 1, :]
        s_ee = s_ee + dee * dee
        s_ne = s_ne + dne * dne
    d_ee = jnp.sqrt(s_ee)
    d_ne = jnp.sqrt(s_ne)

    arow = jnp.dot(r2d, Wdir_all_ref[...],
                   preferred_element_type=f32) + bf_ref[...]
    acol_ee = jnp.dot(r2d, Wdir_ee_ref[...],
                      preferred_element_type=f32)
    acol_ne = jax.lax.dot_general(
        RT2d, Wfne14_ref[...],
        (((0,), (0,)), ((), ())), preferred_element_type=f32)

    acc_ee = (d_ee[:, :, None] * Wf0_ref[0:1, 0:64][None, :, :]
              + arow[:, None, 0:64] - acol_ee[None, :, :])
    e3 = jnp.tanh(acc_ee) * mask_ref[...]
    acc_ne = (d_ne[:, :, None] * Wf0_ref[0:1, 64:96][None, :, :]
              + arow[:, None, 64:96] - acol_ne[None, :, :])
    e_ne3 = jnp.tanh(acc_ne)

    e_ee2 = e3.reshape(_NE * _NE, 64)
    e_ne2 = e_ne3.reshape(_NE * _NN, _S)
    w_ee = jnp.tanh(jnp.dot(e_ee2, BD_ref[...],
                            preferred_element_type=f32))
    w_ne = jnp.tanh(jnp.dot(e_ne2, Wwne_ref[...],
                            preferred_element_type=f32))
    w_ee3 = w_ee.reshape(_NE, _NE, _L * 64)
    w_ne3 = w_ne.reshape(_NE, _NN, _L * _S)

    h = h0_ref[...]
    hs_ne_all = jnp.tanh(jnp.dot(hn_ref[...], Whne_ref[...],
                                 preferred_element_type=f32))
    z_ne_all = jnp.sum(w_ne3 * hs_ne_all[None, :, :], axis=1)

    for l in range(_L):
        hs_ee = jnp.tanh(jnp.dot(h, Whee_ref[l],
                                 preferred_element_type=f32))
        z_ee = jnp.sum(w_ee3[:, :, l * 64:(l + 1) * 64]
                       * hs_ee[None, :, :], axis=1)
        z = jnp.concatenate([z_ne_all[:, l * _S:(l + 1) * _S], z_ee],
                            axis=-1)
        h = h + jnp.tanh(jnp.dot(z, Wu_ref[l],
                                 preferred_element_type=f32) + bu_ref[l])

    out_ref[0] = h


def kernel(r, R, elec_emb, nuc_emb,
           Wfeat_ne, bfeat_ne, Wfeat_same, bfeat_same, Wfeat_anti, bfeat_anti,
           Ww_ne_0, Wh_ne_0, Ww_same_0, Wh_same_0, Ww_anti_0, Wh_anti_0,
           Wu_0, bu_0,
           Ww_ne_1, Wh_ne_1, Ww_same_1, Wh_same_1, Ww_anti_1, Wh_anti_1,
           Wu_1, bu_1,
           Ww_ne_2, Wh_ne_2, Ww_same_2, Wh_same_2, Ww_anti_2, Wh_anti_2,
           Wu_2, bu_2):
    f32 = jnp.float32
    rT = jnp.swapaxes(r, 1, 2)
    RT = jnp.swapaxes(R, 1, 2)

    ii = jnp.arange(_NE)[:, None, None]
    jj = jnp.arange(_NE)[None, :, None]
    ss = jnp.arange(2 * _S)[None, None, :]
    same = ((ii < _NUP) == (jj < _NUP)) & (ii != jj)
    anti = (ii < _NUP) != (jj < _NUP)
    mask = jnp.where(ss < _S, same, anti).astype(f32)

    h0 = jnp.concatenate([jnp.tile(elec_emb[0:1], (_NUP, 1)),
                          jnp.tile(elec_emb[1:2], (_NE - _NUP, 1))])

    Wf0 = jnp.concatenate([Wfeat_same[0:1], Wfeat_anti[0:1],
                           Wfeat_ne[0:1]], axis=1)
    Wdir_all = jnp.concatenate([Wfeat_same[1:4], Wfeat_anti[1:4],
                                Wfeat_ne[1:4]], axis=1)
    Wdir_ee = Wdir_all[:, 0:64]
    Wfne14 = Wfeat_ne[1:4]
    bf = jnp.concatenate([bfeat_same, bfeat_anti,
                          bfeat_ne]).reshape(1, 3 * _S)

    BD = jnp.zeros((64, _L * 64), f32)
    for l, (Ws, Wa) in enumerate([(Ww_same_0, Ww_anti_0),
                                  (Ww_same_1, Ww_anti_1),
                                  (Ww_same_2, Ww_anti_2)]):
        BD = BD.at[0:_S, l * 64:l * 64 + _S].set(Ws)
        BD = BD.at[_S:64, l * 64 + _S:(l + 1) * 64].set(Wa)
    Wwne = jnp.concatenate([Ww_ne_0, Ww_ne_1, Ww_ne_2], axis=1)
    Whne = jnp.concatenate([Wh_ne_0, Wh_ne_1, Wh_ne_2], axis=1)
    Whee = jnp.stack([
        jnp.concatenate([Wh_same_0, Wh_anti_0], axis=1),
        jnp.concatenate([Wh_same_1, Wh_anti_1], axis=1),
        jnp.concatenate([Wh_same_2, Wh_anti_2], axis=1),
    ])
    Wu = jnp.stack([Wu_0, Wu_1, Wu_2])
    bu = jnp.stack([bu_0, bu_1, bu_2]).reshape(_L, 1, _D)

    def const_spec(x):
        nd = x.ndim
        return pl.BlockSpec(x.shape, lambda b, _n=nd: (0,) * _n)

    in_specs = [
        pl.BlockSpec((1, _NE, 3), lambda b: (b, 0, 0)),
        pl.BlockSpec((1, 3, _NE), lambda b: (b, 0, 0)),
        pl.BlockSpec((1, 3, _NN), lambda b: (b, 0, 0)),
    ] + [const_spec(x) for x in (
        mask, h0, nuc_emb,
        Wf0, Wdir_all, Wdir_ee, Wfne14, bf,
        BD, Wwne, Whne, Whee, Wu, bu)]

    out = pl.pallas_call(
        _gnn_body,
        grid=(_B,),
        in_specs=in_specs,
        out_specs=pl.BlockSpec((1, _NE, _D), lambda b: (b, 0, 0)),
        out_shape=jax.ShapeDtypeStruct((_B, _NE, _D), f32),
        compiler_params=pltpu.CompilerParams(
            dimension_semantics=("parallel",)),
    )(r, rT, RT, mask, h0, nuc_emb,
      Wf0, Wdir_all, Wdir_ee, Wfne14, bf,
      BD, Wwne, Whne, Whee, Wu, bu)
    return out


# speedup vs baseline: 2.0783x; 1.1517x over previous
import jax
import jax.numpy as jnp
from jax.experimental import pallas as pl
from jax.experimental.pallas import tpu as pltpu

_B = 128
_NE = 64
_NUP = 32
_NN = 16
_D = 256
_S = 32
_L = 3
_WPG = 2


def _edges(r2d, rT2d, RT2d, mask_ref, Wf0_ref, Wdir_all_ref,
           Wdir_ee_ref, Wfne14_ref, bf_ref):
    f32 = jnp.float32
    s_ee = jnp.full((_NE, _NE), 1e-12, f32)
    s_ne = jnp.full((_NE, _NN), 1e-12, f32)
    for k in range(3):
        col = r2d[:, k:k + 1]
        dee = col - rT2d[k:k + 1, :]
        dne = col - RT2d[k:k + 1, :]
        s_ee = s_ee + dee * dee
        s_ne = s_ne + dne * dne
    d_ee = jnp.sqrt(s_ee)
    d_ne = jnp.sqrt(s_ne)

    arow = jnp.dot(r2d, Wdir_all_ref[...],
                   preferred_element_type=f32) + bf_ref[...]
    acol_ee = jnp.dot(r2d, Wdir_ee_ref[...],
                      preferred_element_type=f32)
    acol_ne = jax.lax.dot_general(
        RT2d, Wfne14_ref[...],
        (((0,), (0,)), ((), ())), preferred_element_type=f32)

    acc_ee = (d_ee[:, :, None] * Wf0_ref[0:1, 0:64][None, :, :]
              + arow[:, None, 0:64] - acol_ee[None, :, :])
    e3 = jnp.tanh(acc_ee) * mask_ref[...]
    acc_ne = (d_ne[:, :, None] * Wf0_ref[0:1, 64:96][None, :, :]
              + arow[:, None, 64:96] - acol_ne[None, :, :])
    e_ne3 = jnp.tanh(acc_ne)
    return e3.reshape(_NE * _NE, 64), e_ne3.reshape(_NE * _NN, _S)


def _gnn_body(r_ref, rT_ref, RT_ref,
              mask_ref, h0_ref, hn_ref,
              Wf0_ref, Wdir_all_ref, Wdir_ee_ref, Wfne14_ref, bf_ref,
              BD_ref, Wwne_ref, Whne_ref, Whee_ref, Wu_ref, bu_ref,
              out_ref):
    f32 = jnp.float32

    ees = []
    nes = []
    for wk in range(_WPG):
        eee, ene = _edges(r_ref[wk], rT_ref[wk], RT_ref[wk],
                          mask_ref, Wf0_ref, Wdir_all_ref,
                          Wdir_ee_ref, Wfne14_ref, bf_ref)
        ees.append(eee)
        nes.append(ene)
    e_ee2 = jnp.concatenate(ees, axis=0)
    e_ne2 = jnp.concatenate(nes, axis=0)

    w_ee = jnp.tanh(jnp.dot(e_ee2, BD_ref[...],
                            preferred_element_type=f32))
    w_ne = jnp.tanh(jnp.dot(e_ne2, Wwne_ref[...],
                            preferred_element_type=f32))
    w_ee4 = w_ee.reshape(_WPG, _NE, _NE, _L * 64)
    w_ne4 = w_ne.reshape(_WPG, _NE, _NN, _L * _S)

    h = h0_ref[...]
    hs_ne_all = jnp.tanh(jnp.dot(hn_ref[...], Whne_ref[...],
                                 preferred_element_type=f32))
    z_nes = [jnp.sum(w_ne4[wk] * hs_ne_all[None, :, :], axis=1)
             for wk in range(_WPG)]

    for l in range(_L):
        hs_ee = jnp.tanh(jnp.dot(h, Whee_ref[l],
                                 preferred_element_type=f32))
        zs = []
        for wk in range(_WPG):
            z_ee = jnp.sum(
                w_ee4[wk][:, :, l * 64:(l + 1) * 64]
                * hs_ee[None, wk * _NE:(wk + 1) * _NE, :],
                axis=1)
            zs.append(jnp.concatenate(
                [z_nes[wk][:, l * _S:(l + 1) * _S], z_ee], axis=-1))
        z = jnp.concatenate(zs, axis=0)
        h = h + jnp.tanh(jnp.dot(z, Wu_ref[l],
                                 preferred_element_type=f32) + bu_ref[l])

    for wk in range(_WPG):
        out_ref[wk] = h[wk * _NE:(wk + 1) * _NE]


def kernel(r, R, elec_emb, nuc_emb,
           Wfeat_ne, bfeat_ne, Wfeat_same, bfeat_same, Wfeat_anti, bfeat_anti,
           Ww_ne_0, Wh_ne_0, Ww_same_0, Wh_same_0, Ww_anti_0, Wh_anti_0,
           Wu_0, bu_0,
           Ww_ne_1, Wh_ne_1, Ww_same_1, Wh_same_1, Ww_anti_1, Wh_anti_1,
           Wu_1, bu_1,
           Ww_ne_2, Wh_ne_2, Ww_same_2, Wh_same_2, Ww_anti_2, Wh_anti_2,
           Wu_2, bu_2):
    f32 = jnp.float32
    rT = jnp.swapaxes(r, 1, 2)
    RT = jnp.swapaxes(R, 1, 2)

    ii = jnp.arange(_NE)[:, None, None]
    jj = jnp.arange(_NE)[None, :, None]
    ss = jnp.arange(2 * _S)[None, None, :]
    same = ((ii < _NUP) == (jj < _NUP)) & (ii != jj)
    anti = (ii < _NUP) != (jj < _NUP)
    mask = jnp.where(ss < _S, same, anti).astype(f32)

    h0 = jnp.concatenate([jnp.tile(elec_emb[0:1], (_NUP, 1)),
                          jnp.tile(elec_emb[1:2], (_NE - _NUP, 1))])
    h0 = jnp.tile(h0, (_WPG, 1))

    Wf0 = jnp.concatenate([Wfeat_same[0:1], Wfeat_anti[0:1],
                           Wfeat_ne[0:1]], axis=1)
    Wdir_all = jnp.concatenate([Wfeat_same[1:4], Wfeat_anti[1:4],
                                Wfeat_ne[1:4]], axis=1)
    Wdir_ee = Wdir_all[:, 0:64]
    Wfne14 = Wfeat_ne[1:4]
    bf = jnp.concatenate([bfeat_same, bfeat_anti,
                          bfeat_ne]).reshape(1, 3 * _S)

    BD = jnp.zeros((64, _L * 64), f32)
    for l, (Ws, Wa) in enumerate([(Ww_same_0, Ww_anti_0),
                                  (Ww_same_1, Ww_anti_1),
                                  (Ww_same_2, Ww_anti_2)]):
        BD = BD.at[0:_S, l * 64:l * 64 + _S].set(Ws)
        BD = BD.at[_S:64, l * 64 + _S:(l + 1) * 64].set(Wa)
    Wwne = jnp.concatenate([Ww_ne_0, Ww_ne_1, Ww_ne_2], axis=1)
    Whne = jnp.concatenate([Wh_ne_0, Wh_ne_1, Wh_ne_2], axis=1)
    Whee = jnp.stack([
        jnp.concatenate([Wh_same_0, Wh_anti_0], axis=1),
        jnp.concatenate([Wh_same_1, Wh_anti_1], axis=1),
        jnp.concatenate([Wh_same_2, Wh_anti_2], axis=1),
    ])
    Wu = jnp.stack([Wu_0, Wu_1, Wu_2])
    bu = jnp.stack([bu_0, bu_1, bu_2]).reshape(_L, 1, _D)

    def const_spec(x):
        nd = x.ndim
        return pl.BlockSpec(x.shape, lambda b, _n=nd: (0,) * _n)

    in_specs = [
        pl.BlockSpec((_WPG, _NE, 3), lambda b: (b, 0, 0)),
        pl.BlockSpec((_WPG, 3, _NE), lambda b: (b, 0, 0)),
        pl.BlockSpec((_WPG, 3, _NN), lambda b: (b, 0, 0)),
    ] + [const_spec(x) for x in (
        mask, h0, nuc_emb,
        Wf0, Wdir_all, Wdir_ee, Wfne14, bf,
        BD, Wwne, Whne, Whee, Wu, bu)]

    out = pl.pallas_call(
        _gnn_body,
        grid=(_B // _WPG,),
        in_specs=in_specs,
        out_specs=pl.BlockSpec((_WPG, _NE, _D), lambda b: (b, 0, 0)),
        out_shape=jax.ShapeDtypeStruct((_B, _NE, _D), f32),
        compiler_params=pltpu.CompilerParams(
            dimension_semantics=("parallel",)),
    )(r, rT, RT, mask, h0, nuc_emb,
      Wf0, Wdir_all, Wdir_ee, Wfne14, bf,
      BD, Wwne, Whne, Whee, Wu, bu)
    return out


# speedup vs baseline: 2.2660x; 1.0903x over previous
import jax
import jax.numpy as jnp
from jax.experimental import pallas as pl
from jax.experimental.pallas import tpu as pltpu

_B = 128
_NE = 64
_NUP = 32
_NN = 16
_D = 256
_S = 32
_L = 3
_WPG = 4


def _edges(r2d, rT2d, RT2d, mask_ref, Wf0_ref, Wdir_all_ref,
           Wdir_ee_ref, Wfne14_ref, bf_ref):
    f32 = jnp.float32
    s_ee = jnp.full((_NE, _NE), 1e-12, f32)
    s_ne = jnp.full((_NE, _NN), 1e-12, f32)
    for k in range(3):
        col = r2d[:, k:k + 1]
        dee = col - rT2d[k:k + 1, :]
        dne = col - RT2d[k:k + 1, :]
        s_ee = s_ee + dee * dee
        s_ne = s_ne + dne * dne
    d_ee = jnp.sqrt(s_ee)
    d_ne = jnp.sqrt(s_ne)

    arow = jnp.dot(r2d, Wdir_all_ref[...],
                   preferred_element_type=f32) + bf_ref[...]
    acol_ee = jnp.dot(r2d, Wdir_ee_ref[...],
                      preferred_element_type=f32)
    acol_ne = jax.lax.dot_general(
        RT2d, Wfne14_ref[...],
        (((0,), (0,)), ((), ())), preferred_element_type=f32)

    acc_ee = (d_ee[:, :, None] * Wf0_ref[0:1, 0:64][None, :, :]
              + arow[:, None, 0:64] - acol_ee[None, :, :])
    e3 = jnp.tanh(acc_ee) * mask_ref[...]
    acc_ne = (d_ne[:, :, None] * Wf0_ref[0:1, 64:96][None, :, :]
              + arow[:, None, 64:96] - acol_ne[None, :, :])
    e_ne3 = jnp.tanh(acc_ne)
    return e3.reshape(_NE * _NE, 64), e_ne3.reshape(_NE * _NN, _S)


def _gnn_body(r_ref, rT_ref, RT_ref,
              mask_ref, h0_ref, hn_ref,
              Wf0_ref, Wdir_all_ref, Wdir_ee_ref, Wfne14_ref, bf_ref,
              BD_ref, Wwne_ref, Whne_ref, Whee_ref, Wu_ref, bu_ref,
              out_ref):
    f32 = jnp.float32

    ees = []
    nes = []
    for wk in range(_WPG):
        eee, ene = _edges(r_ref[wk], rT_ref[wk], RT_ref[wk],
                          mask_ref, Wf0_ref, Wdir_all_ref,
                          Wdir_ee_ref, Wfne14_ref, bf_ref)
        ees.append(eee)
        nes.append(ene)
    e_ee2 = jnp.concatenate(ees, axis=0)
    e_ne2 = jnp.concatenate(nes, axis=0)

    w_ee = jnp.tanh(jnp.dot(e_ee2, BD_ref[...],
                            preferred_element_type=f32))
    w_ne = jnp.tanh(jnp.dot(e_ne2, Wwne_ref[...],
                            preferred_element_type=f32))
    w_ee4 = w_ee.reshape(_WPG, _NE, _NE, _L * 64)
    w_ne4 = w_ne.reshape(_WPG, _NE, _NN, _L * _S)

    h = h0_ref[...]
    hs_ne_all = jnp.tanh(jnp.dot(hn_ref[...], Whne_ref[...],
                                 preferred_element_type=f32))
    z_nes = [jnp.sum(w_ne4[wk] * hs_ne_all[None, :, :], axis=1)
             for wk in range(_WPG)]

    for l in range(_L):
        hs_ee = jnp.tanh(jnp.dot(h, Whee_ref[l],
                                 preferred_element_type=f32))
        zs = []
        for wk in range(_WPG):
            z_ee = jnp.sum(
                w_ee4[wk][:, :, l * 64:(l + 1) * 64]
                * hs_ee[None, wk * _NE:(wk + 1) * _NE, :],
                axis=1)
            zs.append(jnp.concatenate(
                [z_nes[wk][:, l * _S:(l + 1) * _S], z_ee], axis=-1))
        z = jnp.concatenate(zs, axis=0)
        h = h + jnp.tanh(jnp.dot(z, Wu_ref[l],
                                 preferred_element_type=f32) + bu_ref[l])

    for wk in range(_WPG):
        out_ref[wk] = h[wk * _NE:(wk + 1) * _NE]


def kernel(r, R, elec_emb, nuc_emb,
           Wfeat_ne, bfeat_ne, Wfeat_same, bfeat_same, Wfeat_anti, bfeat_anti,
           Ww_ne_0, Wh_ne_0, Ww_same_0, Wh_same_0, Ww_anti_0, Wh_anti_0,
           Wu_0, bu_0,
           Ww_ne_1, Wh_ne_1, Ww_same_1, Wh_same_1, Ww_anti_1, Wh_anti_1,
           Wu_1, bu_1,
           Ww_ne_2, Wh_ne_2, Ww_same_2, Wh_same_2, Ww_anti_2, Wh_anti_2,
           Wu_2, bu_2):
    f32 = jnp.float32
    rT = jnp.swapaxes(r, 1, 2)
    RT = jnp.swapaxes(R, 1, 2)

    ii = jnp.arange(_NE)[:, None, None]
    jj = jnp.arange(_NE)[None, :, None]
    ss = jnp.arange(2 * _S)[None, None, :]
    same = ((ii < _NUP) == (jj < _NUP)) & (ii != jj)
    anti = (ii < _NUP) != (jj < _NUP)
    mask = jnp.where(ss < _S, same, anti).astype(f32)

    h0 = jnp.concatenate([jnp.tile(elec_emb[0:1], (_NUP, 1)),
                          jnp.tile(elec_emb[1:2], (_NE - _NUP, 1))])
    h0 = jnp.tile(h0, (_WPG, 1))

    Wf0 = jnp.concatenate([Wfeat_same[0:1], Wfeat_anti[0:1],
                           Wfeat_ne[0:1]], axis=1)
    Wdir_all = jnp.concatenate([Wfeat_same[1:4], Wfeat_anti[1:4],
                                Wfeat_ne[1:4]], axis=1)
    Wdir_ee = Wdir_all[:, 0:64]
    Wfne14 = Wfeat_ne[1:4]
    bf = jnp.concatenate([bfeat_same, bfeat_anti,
                          bfeat_ne]).reshape(1, 3 * _S)

    BD = jnp.zeros((64, _L * 64), f32)
    for l, (Ws, Wa) in enumerate([(Ww_same_0, Ww_anti_0),
                                  (Ww_same_1, Ww_anti_1),
                                  (Ww_same_2, Ww_anti_2)]):
        BD = BD.at[0:_S, l * 64:l * 64 + _S].set(Ws)
        BD = BD.at[_S:64, l * 64 + _S:(l + 1) * 64].set(Wa)
    Wwne = jnp.concatenate([Ww_ne_0, Ww_ne_1, Ww_ne_2], axis=1)
    Whne = jnp.concatenate([Wh_ne_0, Wh_ne_1, Wh_ne_2], axis=1)
    Whee = jnp.stack([
        jnp.concatenate([Wh_same_0, Wh_anti_0], axis=1),
        jnp.concatenate([Wh_same_1, Wh_anti_1], axis=1),
        jnp.concatenate([Wh_same_2, Wh_anti_2], axis=1),
    ])
    Wu = jnp.stack([Wu_0, Wu_1, Wu_2])
    bu = jnp.stack([bu_0, bu_1, bu_2]).reshape(_L, 1, _D)

    def const_spec(x):
        nd = x.ndim
        return pl.BlockSpec(x.shape, lambda b, _n=nd: (0,) * _n)

    in_specs = [
        pl.BlockSpec((_WPG, _NE, 3), lambda b: (b, 0, 0)),
        pl.BlockSpec((_WPG, 3, _NE), lambda b: (b, 0, 0)),
        pl.BlockSpec((_WPG, 3, _NN), lambda b: (b, 0, 0)),
    ] + [const_spec(x) for x in (
        mask, h0, nuc_emb,
        Wf0, Wdir_all, Wdir_ee, Wfne14, bf,
        BD, Wwne, Whne, Whee, Wu, bu)]

    out = pl.pallas_call(
        _gnn_body,
        grid=(_B // _WPG,),
        in_specs=in_specs,
        out_specs=pl.BlockSpec((_WPG, _NE, _D), lambda b: (b, 0, 0)),
        out_shape=jax.ShapeDtypeStruct((_B, _NE, _D), f32),
        compiler_params=pltpu.CompilerParams(
            dimension_semantics=("parallel",)),
    )(r, rT, RT, mask, h0, nuc_emb,
      Wf0, Wdir_all, Wdir_ee, Wfne14, bf,
      BD, Wwne, Whne, Whee, Wu, bu)
    return out


# speedup vs baseline: 2.3272x; 1.0270x over previous
import jax
import jax.numpy as jnp
from jax.experimental import pallas as pl
from jax.experimental.pallas import tpu as pltpu

_B = 128
_NE = 64
_NUP = 32
_NN = 16
_D = 256
_S = 32
_L = 3
_WPG = 8


def _edges(r2d, rT2d, RT2d, mask_ref, Wf0_ref, Wdir_all_ref,
           Wdir_ee_ref, Wfne14_ref, bf_ref):
    f32 = jnp.float32
    s_ee = jnp.full((_NE, _NE), 1e-12, f32)
    s_ne = jnp.full((_NE, _NN), 1e-12, f32)
    for k in range(3):
        col = r2d[:, k:k + 1]
        dee = col - rT2d[k:k + 1, :]
        dne = col - RT2d[k:k + 1, :]
        s_ee = s_ee + dee * dee
        s_ne = s_ne + dne * dne
    d_ee = jnp.sqrt(s_ee)
    d_ne = jnp.sqrt(s_ne)

    arow = jnp.dot(r2d, Wdir_all_ref[...],
                   preferred_element_type=f32) + bf_ref[...]
    acol_ee = jnp.dot(r2d, Wdir_ee_ref[...],
                      preferred_element_type=f32)
    acol_ne = jax.lax.dot_general(
        RT2d, Wfne14_ref[...],
        (((0,), (0,)), ((), ())), preferred_element_type=f32)

    acc_ee = (d_ee[:, :, None] * Wf0_ref[0:1, 0:64][None, :, :]
              + arow[:, None, 0:64] - acol_ee[None, :, :])
    e3 = jnp.tanh(acc_ee) * mask_ref[...]
    acc_ne = (d_ne[:, :, None] * Wf0_ref[0:1, 64:96][None, :, :]
              + arow[:, None, 64:96] - acol_ne[None, :, :])
    e_ne3 = jnp.tanh(acc_ne)
    return e3.reshape(_NE * _NE, 64), e_ne3.reshape(_NE * _NN, _S)


def _gnn_body(r_ref, rT_ref, RT_ref,
              mask_ref, h0_ref, hn_ref,
              Wf0_ref, Wdir_all_ref, Wdir_ee_ref, Wfne14_ref, bf_ref,
              BD_ref, Wwne_ref, Whne_ref, Whee_ref, Wu_ref, bu_ref,
              out_ref):
    f32 = jnp.float32

    ees = []
    nes = []
    for wk in range(_WPG):
        eee, ene = _edges(r_ref[wk], rT_ref[wk], RT_ref[wk],
                          mask_ref, Wf0_ref, Wdir_all_ref,
                          Wdir_ee_ref, Wfne14_ref, bf_ref)
        ees.append(eee)
        nes.append(ene)
    e_ee2 = jnp.concatenate(ees, axis=0)
    e_ne2 = jnp.concatenate(nes, axis=0)

    w_ee = jnp.tanh(jnp.dot(e_ee2, BD_ref[...],
                            preferred_element_type=f32))
    w_ne = jnp.tanh(jnp.dot(e_ne2, Wwne_ref[...],
                            preferred_element_type=f32))
    w_ee4 = w_ee.reshape(_WPG, _NE, _NE, _L * 64)
    w_ne4 = w_ne.reshape(_WPG, _NE, _NN, _L * _S)

    h = h0_ref[...]
    hs_ne_all = jnp.tanh(jnp.dot(hn_ref[...], Whne_ref[...],
                                 preferred_element_type=f32))
    z_nes = [jnp.sum(w_ne4[wk] * hs_ne_all[None, :, :], axis=1)
             for wk in range(_WPG)]

    for l in range(_L):
        hs_ee = jnp.tanh(jnp.dot(h, Whee_ref[l],
                                 preferred_element_type=f32))
        zs = []
        for wk in range(_WPG):
            z_ee = jnp.sum(
                w_ee4[wk][:, :, l * 64:(l + 1) * 64]
                * hs_ee[None, wk * _NE:(wk + 1) * _NE, :],
                axis=1)
            zs.append(jnp.concatenate(
                [z_nes[wk][:, l * _S:(l + 1) * _S], z_ee], axis=-1))
        z = jnp.concatenate(zs, axis=0)
        h = h + jnp.tanh(jnp.dot(z, Wu_ref[l],
                                 preferred_element_type=f32) + bu_ref[l])

    for wk in range(_WPG):
        out_ref[wk] = h[wk * _NE:(wk + 1) * _NE]


def kernel(r, R, elec_emb, nuc_emb,
           Wfeat_ne, bfeat_ne, Wfeat_same, bfeat_same, Wfeat_anti, bfeat_anti,
           Ww_ne_0, Wh_ne_0, Ww_same_0, Wh_same_0, Ww_anti_0, Wh_anti_0,
           Wu_0, bu_0,
           Ww_ne_1, Wh_ne_1, Ww_same_1, Wh_same_1, Ww_anti_1, Wh_anti_1,
           Wu_1, bu_1,
           Ww_ne_2, Wh_ne_2, Ww_same_2, Wh_same_2, Ww_anti_2, Wh_anti_2,
           Wu_2, bu_2):
    f32 = jnp.float32
    rT = jnp.swapaxes(r, 1, 2)
    RT = jnp.swapaxes(R, 1, 2)

    ii = jnp.arange(_NE)[:, None, None]
    jj = jnp.arange(_NE)[None, :, None]
    ss = jnp.arange(2 * _S)[None, None, :]
    same = ((ii < _NUP) == (jj < _NUP)) & (ii != jj)
    anti = (ii < _NUP) != (jj < _NUP)
    mask = jnp.where(ss < _S, same, anti).astype(f32)

    h0 = jnp.concatenate([jnp.tile(elec_emb[0:1], (_NUP, 1)),
                          jnp.tile(elec_emb[1:2], (_NE - _NUP, 1))])
    h0 = jnp.tile(h0, (_WPG, 1))

    Wf0 = jnp.concatenate([Wfeat_same[0:1], Wfeat_anti[0:1],
                           Wfeat_ne[0:1]], axis=1)
    Wdir_all = jnp.concatenate([Wfeat_same[1:4], Wfeat_anti[1:4],
                                Wfeat_ne[1:4]], axis=1)
    Wdir_ee = Wdir_all[:, 0:64]
    Wfne14 = Wfeat_ne[1:4]
    bf = jnp.concatenate([bfeat_same, bfeat_anti,
                          bfeat_ne]).reshape(1, 3 * _S)

    BD = jnp.zeros((64, _L * 64), f32)
    for l, (Ws, Wa) in enumerate([(Ww_same_0, Ww_anti_0),
                                  (Ww_same_1, Ww_anti_1),
                                  (Ww_same_2, Ww_anti_2)]):
        BD = BD.at[0:_S, l * 64:l * 64 + _S].set(Ws)
        BD = BD.at[_S:64, l * 64 + _S:(l + 1) * 64].set(Wa)
    Wwne = jnp.concatenate([Ww_ne_0, Ww_ne_1, Ww_ne_2], axis=1)
    Whne = jnp.concatenate([Wh_ne_0, Wh_ne_1, Wh_ne_2], axis=1)
    Whee = jnp.stack([
        jnp.concatenate([Wh_same_0, Wh_anti_0], axis=1),
        jnp.concatenate([Wh_same_1, Wh_anti_1], axis=1),
        jnp.concatenate([Wh_same_2, Wh_anti_2], axis=1),
    ])
    Wu = jnp.stack([Wu_0, Wu_1, Wu_2])
    bu = jnp.stack([bu_0, bu_1, bu_2]).reshape(_L, 1, _D)

    def const_spec(x):
        nd = x.ndim
        return pl.BlockSpec(x.shape, lambda b, _n=nd: (0,) * _n)

    in_specs = [
        pl.BlockSpec((_WPG, _NE, 3), lambda b: (b, 0, 0)),
        pl.BlockSpec((_WPG, 3, _NE), lambda b: (b, 0, 0)),
        pl.BlockSpec((_WPG, 3, _NN), lambda b: (b, 0, 0)),
    ] + [const_spec(x) for x in (
        mask, h0, nuc_emb,
        Wf0, Wdir_all, Wdir_ee, Wfne14, bf,
        BD, Wwne, Whne, Whee, Wu, bu)]

    out = pl.pallas_call(
        _gnn_body,
        grid=(_B // _WPG,),
        in_specs=in_specs,
        out_specs=pl.BlockSpec((_WPG, _NE, _D), lambda b: (b, 0, 0)),
        out_shape=jax.ShapeDtypeStruct((_B, _NE, _D), f32),
        compiler_params=pltpu.CompilerParams(
            dimension_semantics=("parallel",)),
    )(r, rT, RT, mask, h0, nuc_emb,
      Wf0, Wdir_all, Wdir_ee, Wfne14, bf,
      BD, Wwne, Whne, Whee, Wu, bu)
    return out
